# Initial kernel scaffold; baseline (speedup 1.0000x reference)
import jax
import jax.numpy as jnp
from jax.experimental import pallas as pl


def kernel(x1, x2, indices, bn_gamma, bn_beta, bn_mean, bn_var, w0, b0, w_sc, b_sc, w_c1, b_c1, w_c2, b_c2):
    raise NotImplementedError("write your pallas kernel here")


# speedup vs baseline: 1.1073x; 1.1073x over previous
import functools

import jax
import jax.numpy as jnp
from jax.experimental import pallas as pl
from jax.experimental.pallas import tpu as pltpu

_F32 = jnp.float32
_BF16 = jnp.bfloat16


def _lrelu(x, slope):
    return jnp.where(x > 0, x, slope * x)


def _block_kernel(x_ref, idx_ref, scale_ref, shift_ref,
                  w0_ref, b0_ref, w5_ref, b5_ref, w2_ref, b2_ref,
                  o_ref, act2_s, s1_s, uc2_s, fc2_s,
                  *, H, W, Cin, Cout):
    HO, WO = 2 * H, 2 * W
    Wp1, Wp2 = W + 2, WO + 4
    M1, M2 = H * Wp1, HO * Wp2
    LX = (H + 3) * Wp1

    a = x_ref[0] * scale_ref[...] + shift_ref[...]
    a = _lrelu(a, 0.1)
    rr = jax.lax.broadcasted_iota(jnp.int32, (LX, 1), 0)
    hh, ww = rr // Wp1, rr % Wp1
    inside = (hh >= 1) & (hh <= H) & (ww >= 1) & (ww <= W)
    a = jnp.where(inside, a, 0.0).astype(_BF16)
    a_nxt = jnp.concatenate([a[1:], jnp.zeros((1, Cin), _BF16)], axis=0)
    act2_s[...] = jnp.concatenate([a, a_nxt], axis=1)

    acc1 = jnp.zeros((M1, Cout), _F32) + b0_ref[...]
    for dy in range(3):
        for j in range(2):
            acc1 = acc1 + jnp.dot(act2_s[pl.ds(dy * Wp1 + 2 * j, M1), :],
                                  w0_ref[2 * dy + j],
                                  preferred_element_type=_F32)
    s1_s[...] = _lrelu(acc1, 0.01).astype(_BF16)

    uc2_s[pl.ds(0, 2 * Wp2), :] = jnp.zeros((2 * Wp2, 2 * Cout), _BF16)
    uc2_s[pl.ds((HO + 2) * Wp2, 3 * Wp2), :] = (
        jnp.zeros((3 * Wp2, 2 * Cout), _BF16))

    two_w = 2 * jax.lax.broadcasted_iota(jnp.int32, (W, Cout), 0)
    z1 = jnp.zeros((1, Cout), _BF16)
    z2 = jnp.zeros((2, Cout), _BF16)
    z3 = jnp.zeros((3, Cout), _BF16)
    z4 = jnp.zeros((4, Cout), _BF16)
    for h in range(H):
        v = s1_s[pl.ds(h * Wp1, W), :]
        irow = idx_ref[0, pl.ds(h * W, W), :]
        base = 2 * h * WO + two_w
        rows = []
        for ry in range(2):
            v0 = jnp.where(irow == base + ry * WO, v, z1)
            v1 = jnp.where(irow == base + ry * WO + 1, v, z1)
            rows.append(jnp.stack([v0, v1], axis=1).reshape(2 * W, Cout))
        first = jnp.concatenate([z2, rows[0], z4, rows[1], z2], axis=0)
        second = jnp.concatenate([z1, rows[0], z4, rows[1], z3], axis=0)
        uc2_s[pl.ds((2 * h + 2) * Wp2, 2 * Wp2), :] = (
            jnp.concatenate([first, second], axis=1))

    acc5 = jnp.zeros((M2, 2 * Cout), _F32) + b5_ref[...]
    for dy in range(5):
        for j in range(3):
            acc5 = acc5 + jnp.dot(uc2_s[pl.ds(dy * Wp2 + 2 * j, M2), :],
                                  w5_ref[3 * dy + j],
                                  preferred_element_type=_F32)
    sc = acc5[:, :Cout]
    f1 = _lrelu(acc5[:, Cout:], 0.01)
    qq = jax.lax.broadcasted_iota(jnp.int32, (M2, 1), 0)
    f1 = jnp.where(qq % Wp2 < WO, f1, 0.0).astype(_BF16)

    f1_nxt = jnp.concatenate([f1[1:], jnp.zeros((1, Cout), _BF16)], axis=0)
    fc2_s[pl.ds(0, Wp2), :] = jnp.zeros((Wp2, 2 * Cout), _BF16)
    fc2_s[pl.ds(Wp2, 1), :] = jnp.concatenate([z1, f1[0:1]], axis=1)
    fc2_s[pl.ds(Wp2 + 1, M2), :] = jnp.concatenate([f1, f1_nxt], axis=1)
    fc2_s[pl.ds(Wp2 + 1 + M2, 2 * Wp2 - 1), :] = (
        jnp.zeros((2 * Wp2 - 1, 2 * Cout), _BF16))

    out = sc + b2_ref[...]
    for dy in range(3):
        for j in range(2):
            out = out + jnp.dot(fc2_s[pl.ds(dy * Wp2 + 2 * j, M2), :],
                                w2_ref[2 * dy + j],
                                preferred_element_type=_F32)
    o_ref[0] = out


def kernel(x1, x2, indices, bn_gamma, bn_beta, bn_mean, bn_var,
           w0, b0, w_sc, b_sc, w_c1, b_c1, w_c2, b_c2):
    N, Cin, H, W = x1.shape
    Cout = w0.shape[3]
    HO, WO = 2 * H, 2 * W
    Wp1, Wp2 = W + 2, WO + 4
    LX = (H + 3) * Wp1
    M1, M2 = H * Wp1, HO * Wp2
    LU = (HO + 5) * Wp2
    LF = (HO + 3) * Wp2
    HW = H * W

    x1n = jnp.transpose(x1, (0, 2, 3, 1)).astype(_F32)
    x1f = jnp.pad(x1n, ((0, 0), (1, 2), (1, 1), (0, 0))).reshape(N, LX, Cin)
    idxf = (jnp.transpose(indices, (0, 2, 3, 1))
            .astype(jnp.int32).reshape(N, HW, Cout))

    eps = 1e-5
    inv_std = 1.0 / jnp.sqrt(bn_var + eps)
    scale = (bn_gamma * inv_std).reshape(1, Cin)
    shift = (bn_beta - bn_mean * bn_gamma * inv_std).reshape(1, Cin)

    def _pairs(w, taps):
        ks = []
        for dy in range(w.shape[0]):
            for dx in range(0, taps, 2):
                top = w[dy, dx]
                bot = (w[dy, dx + 1] if dx + 1 < taps
                       else jnp.zeros_like(top))
                ks.append(jnp.concatenate([top, bot], axis=0))
        return jnp.stack(ks).astype(_BF16)

    w0k = _pairs(w0, 3)
    w5 = jnp.concatenate([w_sc, w_c1], axis=3)
    w5k = _pairs(w5, 5)
    w2k = _pairs(w_c2, 3)

    kfn = functools.partial(_block_kernel, H=H, W=W, Cin=Cin, Cout=Cout)

    out_flat = pl.pallas_call(
        kfn,
        out_shape=jax.ShapeDtypeStruct((N, M2, Cout), _F32),
        grid_spec=pltpu.PrefetchScalarGridSpec(
            num_scalar_prefetch=0,
            grid=(N,),
            in_specs=[
                pl.BlockSpec((1, LX, Cin), lambda n: (n, 0, 0)),
                pl.BlockSpec((1, HW, Cout), lambda n: (n, 0, 0)),
                pl.BlockSpec((1, Cin), lambda n: (0, 0)),
                pl.BlockSpec((1, Cin), lambda n: (0, 0)),
                pl.BlockSpec((6, 2 * Cin, Cout), lambda n: (0, 0, 0)),
                pl.BlockSpec((1, Cout), lambda n: (0, 0)),
                pl.BlockSpec((15, 2 * Cout, 2 * Cout), lambda n: (0, 0, 0)),
                pl.BlockSpec((1, 2 * Cout), lambda n: (0, 0)),
                pl.BlockSpec((6, 2 * Cout, Cout), lambda n: (0, 0, 0)),
                pl.BlockSpec((1, Cout), lambda n: (0, 0)),
            ],
            out_specs=pl.BlockSpec((1, M2, Cout), lambda n: (n, 0, 0)),
            scratch_shapes=[
                pltpu.VMEM((LX, 2 * Cin), _BF16),
                pltpu.VMEM((M1, Cout), _BF16),
                pltpu.VMEM((LU, 2 * Cout), _BF16),
                pltpu.VMEM((LF, 2 * Cout), _BF16),
            ],
        ),
        compiler_params=pltpu.CompilerParams(dimension_semantics=("parallel",)),
    )(x1f, idxf, scale, shift,
      w0k, b0.reshape(1, Cout).astype(_F32),
      w5k, jnp.concatenate([b_sc, b_c1]).reshape(1, 2 * Cout).astype(_F32),
      w2k, b_c2.reshape(1, Cout).astype(_F32))

    out2 = out_flat.reshape(N, HO, Wp2, Cout)[:, :, :WO, :]
    out2 = jnp.transpose(out2, (0, 3, 1, 2))
    return jnp.concatenate([out2, x2.astype(_F32)], axis=1)


# speedup vs baseline: 1.1383x; 1.0280x over previous
import functools

import jax
import jax.numpy as jnp
from jax.experimental import pallas as pl
from jax.experimental.pallas import tpu as pltpu

_F32 = jnp.float32
_BF16 = jnp.bfloat16


def _lrelu(x, slope):
    return jnp.where(x > 0, x, slope * x)


def _block_kernel(x_ref, idx_ref, x2_ref, cm_ref, scale_ref, shift_ref,
                  w0_ref, b0_ref, w5a_ref, w5b_ref, b5_ref,
                  w2a_ref, w2b_ref, b2_ref,
                  o_ref, act2_s, s1_s, uc4_s, fc4_s,
                  *, H, W, Cin, Cout, C2):
    HO, WO = 2 * H, 2 * W
    Wp1, Wp2 = W + 2, WO + 4
    M1, M2 = H * Wp1, HO * Wp2
    LX = (H + 3) * Wp1
    LF = (HO + 3) * Wp2

    a = x_ref[0] * scale_ref[...] + shift_ref[...]
    a = _lrelu(a, 0.1)
    rr = jax.lax.broadcasted_iota(jnp.int32, (LX, 1), 0)
    hh, ww = rr // Wp1, rr % Wp1
    inside = (hh >= 1) & (hh <= H) & (ww >= 1) & (ww <= W)
    a = jnp.where(inside, a, 0.0).astype(_BF16)
    a_nxt = jnp.concatenate([a[1:], jnp.zeros((1, Cin), _BF16)], axis=0)
    act2_s[...] = jnp.concatenate([a, a_nxt], axis=1)

    acc1 = jnp.zeros((M1, Cout), _F32) + b0_ref[...]
    for dy in range(3):
        for j in range(2):
            acc1 = acc1 + jnp.dot(act2_s[pl.ds(dy * Wp1 + 2 * j, M1), :],
                                  w0_ref[2 * dy + j],
                                  preferred_element_type=_F32)
    s1_s[...] = _lrelu(acc1, 0.01).astype(_BF16)

    uc4_s[pl.ds(0, 2 * Wp2), 0:2 * Cout] = jnp.zeros((2 * Wp2, 2 * Cout), _BF16)
    uc4_s[pl.ds((HO + 2) * Wp2, 3 * Wp2), 0:2 * Cout] = (
        jnp.zeros((3 * Wp2, 2 * Cout), _BF16))
    uc4_s[pl.ds(0, Wp2), 2 * Cout:4 * Cout] = (
        jnp.zeros((Wp2, 2 * Cout), _BF16))
    uc4_s[pl.ds((HO + 1) * Wp2, 4 * Wp2), 2 * Cout:4 * Cout] = (
        jnp.zeros((4 * Wp2, 2 * Cout), _BF16))

    two_w = 2 * jax.lax.broadcasted_iota(jnp.int32, (W, Cout), 0)
    z1 = jnp.zeros((1, Cout), _BF16)
    z2 = jnp.zeros((2, Cout), _BF16)
    z3 = jnp.zeros((3, Cout), _BF16)
    z4 = jnp.zeros((4, Cout), _BF16)
    for h in range(H):
        v = s1_s[pl.ds(h * Wp1, W), :]
        irow = idx_ref[0, pl.ds(h * W, W), :]
        base = 2 * h * WO + two_w
        rows = []
        for ry in range(2):
            v0 = jnp.where(irow == base + ry * WO, v, z1)
            v1 = jnp.where(irow == base + ry * WO + 1, v, z1)
            rows.append(jnp.stack([v0, v1], axis=1).reshape(2 * W, Cout))
        first = jnp.concatenate([z2, rows[0], z4, rows[1], z2], axis=0)
        second = jnp.concatenate([z1, rows[0], z4, rows[1], z3], axis=0)
        blk = jnp.concatenate([first, second], axis=1)
        uc4_s[pl.ds((2 * h + 2) * Wp2, 2 * Wp2), 0:2 * Cout] = blk
        uc4_s[pl.ds((2 * h + 1) * Wp2, 2 * Wp2), 2 * Cout:4 * Cout] = blk

    acc5 = jnp.zeros((M2, 2 * Cout), _F32) + b5_ref[...]
    for p in range(2):
        for j in range(3):
            acc5 = acc5 + jnp.dot(
                uc4_s[pl.ds(2 * p * Wp2 + 2 * j, M2), :],
                w5a_ref[3 * p + j], preferred_element_type=_F32)
    for j in range(3):
        acc5 = acc5 + jnp.dot(
            uc4_s[pl.ds(4 * Wp2 + 2 * j, M2), 0:2 * Cout],
            w5b_ref[j], preferred_element_type=_F32)
    sc = acc5[:, :Cout]
    f1 = (_lrelu(acc5[:, Cout:], 0.01) * cm_ref[...]).astype(_BF16)

    f1_nxt = jnp.concatenate([f1[1:], jnp.zeros((1, Cout), _BF16)], axis=0)
    f2 = jnp.concatenate([f1, f1_nxt], axis=1)
    fc4_s[pl.ds(0, Wp2), 0:2 * Cout] = jnp.zeros((Wp2, 2 * Cout), _BF16)
    fc4_s[pl.ds(Wp2, 1), 0:2 * Cout] = jnp.concatenate([z1, f1[0:1]], axis=1)
    fc4_s[pl.ds(Wp2 + 1, M2), 0:2 * Cout] = f2
    fc4_s[pl.ds(Wp2 + 1 + M2, 2 * Wp2 - 1), 0:2 * Cout] = (
        jnp.zeros((2 * Wp2 - 1, 2 * Cout), _BF16))
    fc4_s[pl.ds(0, 1), 2 * Cout:4 * Cout] = jnp.concatenate([z1, f1[0:1]], axis=1)
    fc4_s[pl.ds(1, M2), 2 * Cout:4 * Cout] = f2
    fc4_s[pl.ds(1 + M2, LF - 1 - M2), 2 * Cout:4 * Cout] = (
        jnp.zeros((LF - 1 - M2, 2 * Cout), _BF16))

    out = sc + b2_ref[...]
    for j in range(2):
        out = out + jnp.dot(fc4_s[pl.ds(2 * j, M2), :], w2a_ref[j],
                            preferred_element_type=_F32)
    for j in range(2):
        out = out + jnp.dot(fc4_s[pl.ds(2 * Wp2 + 2 * j, M2), 0:2 * Cout],
                            w2b_ref[j], preferred_element_type=_F32)

    for y in range(HO):
        o_ref[0, 0:Cout, y, :] = jnp.transpose(out[y * Wp2:y * Wp2 + WO, :])
    o_ref[0, pl.ds(Cout, C2)] = x2_ref[0]


def kernel(x1, x2, indices, bn_gamma, bn_beta, bn_mean, bn_var,
           w0, b0, w_sc, b_sc, w_c1, b_c1, w_c2, b_c2):
    N, Cin, H, W = x1.shape
    C2 = x2.shape[1]
    Cout = w0.shape[3]
    HO, WO = 2 * H, 2 * W
    Wp1, Wp2 = W + 2, WO + 4
    LX = (H + 3) * Wp1
    M1, M2 = H * Wp1, HO * Wp2
    LU = (HO + 5) * Wp2
    LF = (HO + 3) * Wp2
    HW = H * W

    x1n = jnp.transpose(x1, (0, 2, 3, 1)).astype(_F32)
    x1f = jnp.pad(x1n, ((0, 0), (1, 2), (1, 1), (0, 0))).reshape(N, LX, Cin)
    idxf = (jnp.transpose(indices, (0, 2, 3, 1))
            .astype(jnp.int32).reshape(N, HW, Cout))

    eps = 1e-5
    inv_std = 1.0 / jnp.sqrt(bn_var + eps)
    scale = (bn_gamma * inv_std).reshape(1, Cin)
    shift = (bn_beta - bn_mean * bn_gamma * inv_std).reshape(1, Cin)

    cm = (jnp.arange(M2, dtype=jnp.int32) % Wp2 < WO).astype(_F32).reshape(M2, 1)

    def _cat(ws):
        return jnp.concatenate(ws, axis=0).astype(_BF16)

    w0k = jnp.stack([_cat([w0[dy, 0], w0[dy, 1]]) if j == 0
                     else _cat([w0[dy, 2], jnp.zeros_like(w0[dy, 2])])
                     for dy in range(3) for j in range(2)])
    w5 = jnp.concatenate([w_sc, w_c1], axis=3)
    z5 = jnp.zeros_like(w5[0, 0])
    w5a = jnp.stack([
        _cat([w5[2 * p, 2 * j], w5[2 * p, 2 * j + 1],
              w5[2 * p + 1, 2 * j], w5[2 * p + 1, 2 * j + 1]]) if j < 2
        else _cat([w5[2 * p, 4], z5, w5[2 * p + 1, 4], z5])
        for p in range(2) for j in range(3)])
    w5b = jnp.stack([_cat([w5[4, 0], w5[4, 1]]),
                     _cat([w5[4, 2], w5[4, 3]]),
                     _cat([w5[4, 4], z5])])
    z2c = jnp.zeros_like(w_c2[0, 0])
    w2a = jnp.stack([_cat([w_c2[0, 0], w_c2[0, 1], w_c2[1, 0], w_c2[1, 1]]),
                     _cat([w_c2[0, 2], z2c, w_c2[1, 2], z2c])])
    w2b = jnp.stack([_cat([w_c2[2, 0], w_c2[2, 1]]),
                     _cat([w_c2[2, 2], z2c])])

    kfn = functools.partial(_block_kernel, H=H, W=W, Cin=Cin, Cout=Cout, C2=C2)

    return pl.pallas_call(
        kfn,
        out_shape=jax.ShapeDtypeStruct((N, Cout + C2, HO, WO), _F32),
        grid_spec=pltpu.PrefetchScalarGridSpec(
            num_scalar_prefetch=0,
            grid=(N,),
            in_specs=[
                pl.BlockSpec((1, LX, Cin), lambda n: (n, 0, 0)),
                pl.BlockSpec((1, HW, Cout), lambda n: (n, 0, 0)),
                pl.BlockSpec((1, C2, HO, WO), lambda n: (n, 0, 0, 0)),
                pl.BlockSpec((M2, 1), lambda n: (0, 0)),
                pl.BlockSpec((1, Cin), lambda n: (0, 0)),
                pl.BlockSpec((1, Cin), lambda n: (0, 0)),
                pl.BlockSpec((6, 2 * Cin, Cout), lambda n: (0, 0, 0)),
                pl.BlockSpec((1, Cout), lambda n: (0, 0)),
                pl.BlockSpec((6, 4 * Cout, 2 * Cout), lambda n: (0, 0, 0)),
                pl.BlockSpec((3, 2 * Cout, 2 * Cout), lambda n: (0, 0, 0)),
                pl.BlockSpec((1, 2 * Cout), lambda n: (0, 0)),
                pl.BlockSpec((2, 4 * Cout, Cout), lambda n: (0, 0, 0)),
                pl.BlockSpec((2, 2 * Cout, Cout), lambda n: (0, 0, 0)),
                pl.BlockSpec((1, Cout), lambda n: (0, 0)),
            ],
            out_specs=pl.BlockSpec((1, Cout + C2, HO, WO),
                                   lambda n: (n, 0, 0, 0)),
            scratch_shapes=[
                pltpu.VMEM((LX, 2 * Cin), _BF16),
                pltpu.VMEM((M1, Cout), _BF16),
                pltpu.VMEM((LU, 4 * Cout), _BF16),
                pltpu.VMEM((LF, 4 * Cout), _BF16),
            ],
        ),
        compiler_params=pltpu.CompilerParams(dimension_semantics=("parallel",)),
    )(x1f, idxf, x2.astype(_F32), cm, scale, shift,
      w0k, b0.reshape(1, Cout).astype(_F32),
      w5a, w5b,
      jnp.concatenate([b_sc, b_c1]).reshape(1, 2 * Cout).astype(_F32),
      w2a, w2b, b_c2.reshape(1, Cout).astype(_F32))


# speedup vs baseline: 1.3072x; 1.1484x over previous
import functools

import jax
import jax.numpy as jnp
from jax.experimental import pallas as pl
from jax.experimental.pallas import tpu as pltpu

_F32 = jnp.float32
_BF16 = jnp.bfloat16


def _lrelu(x, slope):
    return jnp.where(x > 0, x, slope * x)


def _block_kernel(x_ref, idx_ref, x2_ref, cm_ref, scale_ref, shift_ref,
                  w0_ref, b0_ref, w5_ref, b5_ref, w2_ref, b2_ref,
                  o_ref, act_s, s1_s, uc_s, fc_s,
                  *, H, W, Cin, Cout, C2):
    HO, WO = 2 * H, 2 * W
    Wp1, Wp2 = W + 8, WO + 8
    M1, M2 = H * Wp1, HO * Wp2
    LX = (H + 3) * Wp1
    S1, S5, S2 = 16, Wp2, Wp2

    a = x_ref[0] * scale_ref[...] + shift_ref[...]
    a = _lrelu(a, 0.1)
    rr = jax.lax.broadcasted_iota(jnp.int32, (LX, 1), 0)
    hh, ww = rr // Wp1, rr % Wp1
    inside = (hh >= 1) & (hh <= H) & (ww >= 1) & (ww <= W)
    a = jnp.where(inside, a, 0.0).astype(_BF16)
    for dx in range(3):
        act_s[pl.ds(S1 - dx, LX), dx * Cin:(dx + 1) * Cin] = a

    acc1 = jnp.zeros((M1, Cout), _F32) + b0_ref[...]
    for dy in range(3):
        acc1 = acc1 + jnp.dot(act_s[pl.ds(S1 + dy * Wp1, M1), :], w0_ref[dy],
                              preferred_element_type=_F32)
    s1_s[...] = _lrelu(acc1, 0.01).astype(_BF16)

    z1 = jnp.zeros((1, Cout), _BF16)
    z2 = jnp.zeros((2, Cout), _BF16)
    z4 = jnp.zeros((4, Cout), _BF16)
    z6 = jnp.zeros((6, Cout), _BF16)
    for dx in range(5):
        tz = 2 * Wp2 + (dx if dx < 3 else 0)
        uc_s[pl.ds(S5 - dx, tz), dx * Cout:(dx + 1) * Cout] = (
            jnp.zeros((tz, Cout), _BF16))
        lo = (HO + 2) * Wp2 + S5 - (dx if dx < 3 else 8)
        uc_s[pl.ds(lo, 3 * Wp2), dx * Cout:(dx + 1) * Cout] = (
            jnp.zeros((3 * Wp2, Cout), _BF16))

    for h in range(H):
        v = s1_s[pl.ds(h * Wp1, W), :]
        irow = idx_ref[0, pl.ds(h * W, W), :]
        sel = ((irow // WO) & 1) * 2 + (irow & 1)
        rows = []
        for ry in range(2):
            v0 = jnp.where(sel == 2 * ry, v, z1)
            v1 = jnp.where(sel == 2 * ry + 1, v, z1)
            rows.append(jnp.stack([v0, v1], axis=1).reshape(2 * W, Cout))
        base = (2 * h + 2) * Wp2 + S5
        def _shifted(lead, tail):
            ps = [jnp.zeros((lead, Cout), _BF16), rows[0],
                  jnp.zeros((8, Cout), _BF16), rows[1],
                  jnp.zeros((tail, Cout), _BF16)]
            return jnp.concatenate([p for p in ps if p.shape[0]], axis=0)

        lo = jnp.concatenate([_shifted(2 - dx, 6 + dx) for dx in range(3)],
                             axis=1)
        uc_s[pl.ds(base, 2 * Wp2), 0:3 * Cout] = lo
        hi4 = jnp.concatenate([_shifted(10 - dx, dx - 2) for dx in (3, 4)],
                              axis=1)
        uc_s[pl.ds(base - 8, 2 * Wp2), 3 * Cout:5 * Cout] = hi4

    acc5 = jnp.zeros((M2, 2 * Cout), _F32) + b5_ref[...]
    for dy in range(5):
        acc5 = acc5 + jnp.dot(uc_s[pl.ds(S5 + dy * Wp2, M2), :], w5_ref[dy],
                              preferred_element_type=_F32)
    sc = acc5[:, :Cout]
    f1 = (_lrelu(acc5[:, Cout:], 0.01) * cm_ref[...]).astype(_BF16)

    for dx in range(3):
        q0 = S2 + Wp2 + 1 - dx
        lanes = dx * Cout
        fc_s[pl.ds(q0, M2), lanes:lanes + Cout] = f1
        fc_s[pl.ds(S2 - dx, Wp2 + 1), lanes:lanes + Cout] = (
            jnp.zeros((Wp2 + 1, Cout), _BF16))
        hi = S2 + 2 * Wp2 + M2
        fc_s[pl.ds(q0 + M2, hi - q0 - M2), lanes:lanes + Cout] = (
            jnp.zeros((hi - q0 - M2, Cout), _BF16))

    out = sc + b2_ref[...]
    for dy in range(3):
        out = out + jnp.dot(fc_s[pl.ds(S2 + dy * Wp2, M2), :], w2_ref[dy],
                            preferred_element_type=_F32)

    for y in range(HO):
        o_ref[0, 0:Cout, y, :] = jnp.transpose(out[y * Wp2:y * Wp2 + WO, :])
    o_ref[0, pl.ds(Cout, C2)] = x2_ref[0]


def kernel(x1, x2, indices, bn_gamma, bn_beta, bn_mean, bn_var,
           w0, b0, w_sc, b_sc, w_c1, b_c1, w_c2, b_c2):
    N, Cin, H, W = x1.shape
    C2 = x2.shape[1]
    Cout = w0.shape[3]
    HO, WO = 2 * H, 2 * W
    Wp1, Wp2 = W + 8, WO + 8
    LX = (H + 3) * Wp1
    M1, M2 = H * Wp1, HO * Wp2
    HW = H * W

    x1n = jnp.transpose(x1, (0, 2, 3, 1)).astype(_F32)
    x1f = jnp.pad(x1n, ((0, 0), (1, 2), (1, 7), (0, 0))).reshape(N, LX, Cin)
    idxf = (jnp.transpose(indices, (0, 2, 3, 1))
            .astype(jnp.int32).reshape(N, HW, Cout))

    eps = 1e-5
    inv_std = 1.0 / jnp.sqrt(bn_var + eps)
    scale = (bn_gamma * inv_std).reshape(1, Cin)
    shift = (bn_beta - bn_mean * bn_gamma * inv_std).reshape(1, Cin)

    cm = (jnp.arange(M2, dtype=jnp.int32) % Wp2 < WO).astype(_F32).reshape(M2, 1)

    def _rowstack(w):
        taps = w.shape[0]
        return jnp.stack(
            [jnp.concatenate([w[dy, dx] for dx in range(taps)], axis=0)
             for dy in range(taps)]).astype(_BF16)

    w0k = _rowstack(w0)
    w5 = jnp.concatenate([w_sc, w_c1], axis=3)
    w5k = _rowstack(w5)
    w2k = _rowstack(w_c2)

    kfn = functools.partial(_block_kernel, H=H, W=W, Cin=Cin, Cout=Cout, C2=C2)

    LX3 = LX + 16
    LU5 = (HO + 7) * Wp2
    LF3 = (HO + 5) * Wp2

    return pl.pallas_call(
        kfn,
        out_shape=jax.ShapeDtypeStruct((N, Cout + C2, HO, WO), _F32),
        grid_spec=pltpu.PrefetchScalarGridSpec(
            num_scalar_prefetch=0,
            grid=(N,),
            in_specs=[
                pl.BlockSpec((1, LX, Cin), lambda n: (n, 0, 0)),
                pl.BlockSpec((1, HW, Cout), lambda n: (n, 0, 0)),
                pl.BlockSpec((1, C2, HO, WO), lambda n: (n, 0, 0, 0)),
                pl.BlockSpec((M2, 1), lambda n: (0, 0)),
                pl.BlockSpec((1, Cin), lambda n: (0, 0)),
                pl.BlockSpec((1, Cin), lambda n: (0, 0)),
                pl.BlockSpec((3, 3 * Cin, Cout), lambda n: (0, 0, 0)),
                pl.BlockSpec((1, Cout), lambda n: (0, 0)),
                pl.BlockSpec((5, 5 * Cout, 2 * Cout), lambda n: (0, 0, 0)),
                pl.BlockSpec((1, 2 * Cout), lambda n: (0, 0)),
                pl.BlockSpec((3, 3 * Cout, Cout), lambda n: (0, 0, 0)),
                pl.BlockSpec((1, Cout), lambda n: (0, 0)),
            ],
            out_specs=pl.BlockSpec((1, Cout + C2, HO, WO),
                                   lambda n: (n, 0, 0, 0)),
            scratch_shapes=[
                pltpu.VMEM((LX3, 3 * Cin), _BF16),
                pltpu.VMEM((M1, Cout), _BF16),
                pltpu.VMEM((LU5, 5 * Cout), _BF16),
                pltpu.VMEM((LF3, 3 * Cout), _BF16),
            ],
        ),
        compiler_params=pltpu.CompilerParams(
            dimension_semantics=("parallel",)),
    )(x1f, idxf, x2.astype(_F32), cm, scale, shift,
      w0k, b0.reshape(1, Cout).astype(_F32),
      w5k, jnp.concatenate([b_sc, b_c1]).reshape(1, 2 * Cout).astype(_F32),
      w2k, b_c2.reshape(1, Cout).astype(_F32))
